# Initial kernel scaffold; baseline (speedup 1.0000x reference)
import jax
import jax.numpy as jnp
from jax.experimental import pallas as pl


def kernel(x, table):
    raise NotImplementedError("write your pallas kernel here")


# speedup vs baseline: 1.1029x; 1.1029x over previous
import functools

import jax
import jax.numpy as jnp
from jax import lax
from jax.experimental import pallas as pl
from jax.experimental.pallas import tpu as pltpu
from jax.experimental.pallas import tpu_sc as plsc

_NC = 2
_NS = 16
_NW = _NC * _NS

_D = 32
_IDX_W = 128
_GATHERS_PER_BLK = 8
_BLK = _IDX_W * _GATHERS_PER_BLK


def _lookup(x2d, table, *, b_total):
    rows_per_w = b_total // _NW
    idx_rows_per_w = rows_per_w // _IDX_W
    n_blk = rows_per_w // _BLK

    mesh = plsc.VectorSubcoreMesh(core_axis_name="c", subcore_axis_name="s")

    @functools.partial(
        pl.kernel,
        out_type=jax.ShapeDtypeStruct((b_total, _D), jnp.float32),
        mesh=mesh,
        scratch_types=[
            pltpu.VMEM((idx_rows_per_w, _IDX_W), jnp.int32),
            pltpu.VMEM((_BLK, _D), jnp.float32),
            pltpu.SemaphoreType.DMA,
        ],
        compiler_params=pltpu.CompilerParams(use_tc_tiling_on_sc=False),
    )
    def body(x_hbm, table_hbm, out_hbm, idx_v, rows_v, sem):
        wid = lax.axis_index("s") * _NC + lax.axis_index("c")
        pltpu.sync_copy(x_hbm.at[pl.ds(wid * idx_rows_per_w, idx_rows_per_w)], idx_v)
        out_base = wid * rows_per_w

        def blk(i, carry):
            copies = []
            for j in range(_GATHERS_PER_BLK):
                copies.append(
                    pltpu.async_copy(
                        table_hbm.at[idx_v.at[i * _GATHERS_PER_BLK + j]],
                        rows_v.at[pl.ds(j * _IDX_W, _IDX_W)],
                        sem,
                    )
                )
            for c in copies:
                c.wait()
            pltpu.sync_copy(rows_v, out_hbm.at[pl.ds(out_base + i * _BLK, _BLK)])
            return carry

        lax.fori_loop(0, n_blk, blk, 0)

    return body(x2d, table)


def kernel(x, table):
    b, s = x.shape
    b_total = b * s
    x2d = x.reshape(b_total // _IDX_W, _IDX_W).astype(jnp.int32)
    out = _lookup(x2d, table, b_total=b_total)
    return out.reshape(b, s, _D)


# speedup vs baseline: 1.1136x; 1.0097x over previous
import functools

import jax
import jax.numpy as jnp
from jax import lax
from jax.experimental import pallas as pl
from jax.experimental.pallas import tpu as pltpu
from jax.experimental.pallas import tpu_sc as plsc

_NC = 2
_NS = 16
_NW = _NC * _NS

_D = 32
_BLK = 1024


def _lookup(x1d, table, *, b_total):
    rows_per_w = b_total // _NW
    n_blk = rows_per_w // _BLK

    mesh = plsc.VectorSubcoreMesh(core_axis_name="c", subcore_axis_name="s")

    @functools.partial(
        pl.kernel,
        out_type=jax.ShapeDtypeStruct((b_total, _D), jnp.float32),
        mesh=mesh,
        scratch_types=[
            pltpu.VMEM((rows_per_w,), jnp.int32),
            pltpu.VMEM((2, _BLK, _D), jnp.float32),
            pltpu.SemaphoreType.DMA,
            pltpu.SemaphoreType.DMA,
            pltpu.SemaphoreType.DMA,
            pltpu.SemaphoreType.DMA,
        ],
        compiler_params=pltpu.CompilerParams(use_tc_tiling_on_sc=False),
    )
    def body(x_hbm, table_hbm, out_hbm, idx_v, rows_v, g0, g1, s0, s1):
        wid = lax.axis_index("s") * _NC + lax.axis_index("c")
        pltpu.sync_copy(x_hbm.at[pl.ds(wid * rows_per_w, rows_per_w)], idx_v)
        out_base = wid * rows_per_w

        gsem = (g0, g1)
        ssem = (s0, s1)

        def gather(i):
            return pltpu.async_copy(
                table_hbm.at[idx_v.at[pl.ds(i * _BLK, _BLK)]],
                rows_v.at[i % 2],
                gsem[i % 2],
            )

        gd = [None, None]
        sd = [None, None]
        gd[0] = gather(0)
        for i in range(n_blk):
            b = i % 2
            nb = (i + 1) % 2
            if i + 1 < n_blk:
                if sd[nb] is not None:
                    sd[nb].wait()
                gd[nb] = gather(i + 1)
            gd[b].wait()
            sd[b] = pltpu.async_copy(
                rows_v.at[b],
                out_hbm.at[pl.ds(out_base + i * _BLK, _BLK)],
                ssem[b],
            )
        for d in sd:
            if d is not None:
                d.wait()

    return body(x1d, table)


def kernel(x, table):
    b, s = x.shape
    b_total = b * s
    x1d = x.reshape(b_total).astype(jnp.int32)
    out = _lookup(x1d, table, b_total=b_total)
    return out.reshape(b, s, _D)


# speedup vs baseline: 1.8016x; 1.6179x over previous
import functools

import jax
import jax.numpy as jnp
from jax import lax
from jax.experimental import pallas as pl
from jax.experimental.pallas import tpu as pltpu
from jax.experimental.pallas import tpu_sc as plsc

_NC = 2
_NS = 16
_NW = _NC * _NS

_D = 32
_BB = 16


def _lookup(x, table):
    b, s = x.shape
    b_per_w = b // _NW
    n_blk = b_per_w // _BB

    mesh = plsc.VectorSubcoreMesh(core_axis_name="c", subcore_axis_name="s")

    @functools.partial(
        pl.kernel,
        out_type=jax.ShapeDtypeStruct((b, s, _D), jnp.float32),
        mesh=mesh,
        scratch_types=[
            pltpu.VMEM((b_per_w, s), jnp.int32),
            pltpu.VMEM((2, _BB, s, _D), jnp.float32),
            pltpu.SemaphoreType.DMA,
            pltpu.SemaphoreType.DMA,
            pltpu.SemaphoreType.DMA,
            pltpu.SemaphoreType.DMA,
        ],
        compiler_params=pltpu.CompilerParams(use_tc_tiling_on_sc=False),
    )
    def body(x_hbm, table_hbm, out_hbm, idx_v, rows_v, g0, g1, s0, s1):
        wid = lax.axis_index("s") * _NC + lax.axis_index("c")
        pltpu.sync_copy(x_hbm.at[pl.ds(wid * b_per_w, b_per_w)], idx_v)
        out_base = wid * b_per_w

        gsem = (g0, g1)
        ssem = (s0, s1)

        def gather(i):
            buf = i % 2
            copies = []
            for j in range(_BB):
                copies.append(
                    pltpu.async_copy(
                        table_hbm.at[idx_v.at[i * _BB + j]],
                        rows_v.at[buf, j],
                        gsem[buf],
                    )
                )
            return copies

        gd = [None, None]
        sd = [None, None]
        gd[0] = gather(0)
        for i in range(n_blk):
            buf = i % 2
            nbuf = (i + 1) % 2
            if i + 1 < n_blk:
                if sd[nbuf] is not None:
                    sd[nbuf].wait()
                gd[nbuf] = gather(i + 1)
            for c in gd[buf]:
                c.wait()
            sd[buf] = pltpu.async_copy(
                rows_v.at[buf],
                out_hbm.at[pl.ds(out_base + i * _BB, _BB)],
                ssem[buf],
            )
        for d in sd:
            if d is not None:
                d.wait()

    return body(x, table)


def kernel(x, table):
    return _lookup(x.astype(jnp.int32), table)
